# Initial kernel scaffold; baseline (speedup 1.0000x reference)
import jax
import jax.numpy as jnp
from jax import lax
from jax.experimental import pallas as pl
from jax.experimental.pallas import tpu as pltpu
from jax.experimental.pallas import tpu_sc as plsc

NG = 256
NN = 64
NF = 9
H = 768
L = 16
NC = 2
NS = 16
NW = NC * NS

GPW = NG // NW
SUB = 8
ROWS = SUB * NF
STEPS = GPW * (NN // SUB)
DCH = 64
DSTEPS = GPW * NN // DCH
CB = H // L


def _body(x2, ind2, outd2, atom_w, in_w, out_w, tok,
          out_flat, deg_flat,
          xidx, din, dout, bufA, bufB, obuf, tokv,
          semA, semB):
    wid = lax.axis_index("s") * NC + lax.axis_index("c")

    pltpu.sync_copy(x2.at[pl.ds(wid * STEPS, STEPS)], xidx)
    pltpu.sync_copy(ind2.at[pl.ds(wid * GPW, GPW)], din)
    pltpu.sync_copy(outd2.at[pl.ds(wid * GPW, GPW)], dout)
    pltpu.sync_copy(tok, tokv)

    def gather(s, buf, sem):
        pltpu.async_copy(atom_w.at[xidx.at[s]], buf, sem)

    def compute_store(s, buf):
        def nbody(n, _):
            base = n * NF
            for j in range(CB):
                c = j * L
                acc = buf[base, pl.ds(c, L)]
                for f in range(1, NF):
                    acc = acc + buf[base + f, pl.ds(c, L)]
                obuf[n, pl.ds(c, L)] = acc
            return 0
        lax.fori_loop(0, SUB, nbody, 0)
        g = wid * GPW + s // 8
        orow = g * (NN + 1) + 1 + (s % 8) * SUB
        pltpu.sync_copy(obuf, out_flat.at[pl.ds(orow, SUB)])

    gather(0, bufA, semA)

    def step2(i, _):
        s0 = i * 2
        pltpu.make_async_copy(atom_w.at[xidx.at[s0]], bufA, semA).wait()
        gather(s0 + 1, bufB, semB)
        compute_store(s0, bufA)
        pltpu.make_async_copy(atom_w.at[xidx.at[s0]], bufB, semB).wait()

        @pl.when(s0 + 2 < STEPS)
        def _():
            gather(s0 + 2, bufA, semA)

        compute_store(s0 + 1, bufB)
        return 0

    lax.fori_loop(0, STEPS // 2, step2, 0)

    def tbody(gl, _):
        g = wid * GPW + gl
        pltpu.sync_copy(tokv, out_flat.at[pl.ds(g * (NN + 1), 1)])
        return 0

    lax.fori_loop(0, GPW, tbody, 0)

    def dbody(c, _):
        pltpu.async_copy(in_w.at[din.at[c]], bufA.at[pl.ds(0, DCH)], semA)
        pltpu.async_copy(out_w.at[dout.at[c]], bufB.at[pl.ds(0, DCH)], semB)
        pltpu.make_async_copy(in_w.at[din.at[c]], bufA.at[pl.ds(0, DCH)],
                              semA).wait()
        pltpu.make_async_copy(out_w.at[dout.at[c]], bufB.at[pl.ds(0, DCH)],
                              semB).wait()

        def rbody(r, _):
            for j in range(CB):
                col = j * L
                bufA[r, pl.ds(col, L)] = (bufA[r, pl.ds(col, L)]
                                          + bufB[r, pl.ds(col, L)])
            return 0

        lax.fori_loop(0, DCH, rbody, 0)
        pltpu.sync_copy(bufA.at[pl.ds(0, DCH)],
                        deg_flat.at[pl.ds(wid * GPW * NN + c * DCH, DCH)])
        return 0

    lax.fori_loop(0, DSTEPS, dbody, 0)


@jax.jit
def kernel(x, in_degree, out_degree, atom_w, in_w, out_w, graph_token):
    x2 = x.astype(jnp.int32).reshape(NW * STEPS, ROWS)
    ind2 = in_degree.astype(jnp.int32).reshape(NG, NN)
    outd2 = out_degree.astype(jnp.int32).reshape(NG, NN)

    kfn = pl.kernel(
        _body,
        out_type=(
            jax.ShapeDtypeStruct((NG * (NN + 1), H), jnp.float32),
            jax.ShapeDtypeStruct((NG * NN, H), jnp.float32),
        ),
        mesh=plsc.VectorSubcoreMesh(core_axis_name="c", subcore_axis_name="s"),
        scratch_types=[
            pltpu.VMEM((STEPS, ROWS), jnp.int32),
            pltpu.VMEM((GPW, NN), jnp.int32),
            pltpu.VMEM((GPW, NN), jnp.int32),
            pltpu.VMEM((ROWS, H), jnp.float32),
            pltpu.VMEM((ROWS, H), jnp.float32),
            pltpu.VMEM((SUB, H), jnp.float32),
            pltpu.VMEM((1, H), jnp.float32),
            pltpu.SemaphoreType.DMA,
            pltpu.SemaphoreType.DMA,
        ],
    )
    out_flat, deg_flat = kfn(x2, ind2, outd2, atom_w, in_w, out_w, graph_token)
    return (out_flat.reshape(NG, NN + 1, H), deg_flat.reshape(NG, NN, H))


# speedup vs baseline: 2.7557x; 2.7557x over previous
import jax
import jax.numpy as jnp
from jax import lax
from jax.experimental import pallas as pl
from jax.experimental.pallas import tpu as pltpu
from jax.experimental.pallas import tpu_sc as plsc

NG = 256
NN = 64
NR = NN + 1
NF = 9
H = 768
L = 16
NC = 2
NS = 16
NW = NC * NS

GPW = NG // NW
RPW = GPW * NR
SUB = 8
ROWS = SUB * NF
STEPS = RPW // SUB
DCH = 64
DSTEPS = GPW * NN // DCH
CB = H // L


def _body(idx9, ind_f, outd_f, atom_w, in_w, out_w, tok,
          out_flat, deg_flat,
          xidx, din, dout, bufA, bufB, obuf, tokv,
          semA, semB):
    wid = lax.axis_index("s") * NC + lax.axis_index("c")

    pltpu.sync_copy(idx9.at[pl.ds(wid * RPW * NF, RPW * NF)], xidx)
    pltpu.sync_copy(ind_f.at[pl.ds(wid * GPW * NN, GPW * NN)], din)
    pltpu.sync_copy(outd_f.at[pl.ds(wid * GPW * NN, GPW * NN)], dout)
    pltpu.sync_copy(tok, tokv)

    def gather(s, buf, sem):
        pltpu.async_copy(atom_w.at[xidx.at[pl.ds(s * ROWS, ROWS)]], buf, sem)

    def wait(s, buf, sem):
        pltpu.make_async_copy(
            atom_w.at[xidx.at[pl.ds(s * ROWS, ROWS)]], buf, sem).wait()

    def compute_store(s, buf):
        def nbody(n, _):
            base = n * NF
            for j in range(CB):
                c = j * L
                acc = buf[base, pl.ds(c, L)]
                for f in range(1, NF):
                    acc = acc + buf[base + f, pl.ds(c, L)]
                obuf[n, pl.ds(c, L)] = acc
            return 0
        lax.fori_loop(0, SUB, nbody, 0)

        @pl.when(s % 8 == 0)
        def _():
            r = s // 8
            def tfill(j, _):
                c = j * L
                obuf[r, pl.ds(c, L)] = tokv[0, pl.ds(c, L)]
                return 0
            lax.fori_loop(0, CB, tfill, 0)

        pltpu.sync_copy(obuf, out_flat.at[pl.ds(wid * RPW + s * SUB, SUB)])

    gather(0, bufA, semA)

    def step2(i, _):
        s0 = i * 2
        wait(s0, bufA, semA)
        gather(s0 + 1, bufB, semB)
        compute_store(s0, bufA)
        wait(s0 + 1, bufB, semB)
        gather(s0 + 2, bufA, semA)
        compute_store(s0 + 1, bufB)
        return 0

    lax.fori_loop(0, (STEPS - 1) // 2, step2, 0)
    wait(STEPS - 1, bufA, semA)
    compute_store(STEPS - 1, bufA)

    def dbody(c, _):
        pltpu.async_copy(in_w.at[din.at[pl.ds(c * DCH, DCH)]],
                         bufA.at[pl.ds(0, DCH)], semA)
        pltpu.async_copy(out_w.at[dout.at[pl.ds(c * DCH, DCH)]],
                         bufB.at[pl.ds(0, DCH)], semB)
        pltpu.make_async_copy(in_w.at[din.at[pl.ds(c * DCH, DCH)]],
                              bufA.at[pl.ds(0, DCH)], semA).wait()
        pltpu.make_async_copy(out_w.at[dout.at[pl.ds(c * DCH, DCH)]],
                              bufB.at[pl.ds(0, DCH)], semB).wait()

        def rbody(r, _):
            for j in range(CB):
                col = j * L
                bufA[r, pl.ds(col, L)] = (bufA[r, pl.ds(col, L)]
                                          + bufB[r, pl.ds(col, L)])
            return 0

        lax.fori_loop(0, DCH, rbody, 0)
        pltpu.sync_copy(bufA.at[pl.ds(0, DCH)],
                        deg_flat.at[pl.ds(wid * GPW * NN + c * DCH, DCH)])
        return 0

    lax.fori_loop(0, DSTEPS, dbody, 0)


@jax.jit
def kernel(x, in_degree, out_degree, atom_w, in_w, out_w, graph_token):
    x3 = x.astype(jnp.int32).reshape(NG, NN, NF)
    idx9 = jnp.pad(x3, ((0, 0), (1, 0), (0, 0))).reshape(NG * NR * NF)
    ind_f = in_degree.astype(jnp.int32).reshape(NG * NN)
    outd_f = out_degree.astype(jnp.int32).reshape(NG * NN)

    kfn = pl.kernel(
        _body,
        out_type=(
            jax.ShapeDtypeStruct((NG * NR, H), jnp.float32),
            jax.ShapeDtypeStruct((NG * NN, H), jnp.float32),
        ),
        mesh=plsc.VectorSubcoreMesh(core_axis_name="c", subcore_axis_name="s"),
        scratch_types=[
            pltpu.VMEM((RPW * NF,), jnp.int32),
            pltpu.VMEM((GPW * NN,), jnp.int32),
            pltpu.VMEM((GPW * NN,), jnp.int32),
            pltpu.VMEM((ROWS, H), jnp.float32),
            pltpu.VMEM((ROWS, H), jnp.float32),
            pltpu.VMEM((SUB, H), jnp.float32),
            pltpu.VMEM((1, H), jnp.float32),
            pltpu.SemaphoreType.DMA,
            pltpu.SemaphoreType.DMA,
        ],
    )
    out_flat, deg_flat = kfn(idx9, ind_f, outd_f, atom_w, in_w, out_w,
                             graph_token)
    return (out_flat.reshape(NG, NR, H), deg_flat.reshape(NG, NN, H))


# speedup vs baseline: 3.2267x; 1.1709x over previous
import jax
import jax.numpy as jnp
from jax import lax
from jax.experimental import pallas as pl
from jax.experimental.pallas import tpu as pltpu
from jax.experimental.pallas import tpu_sc as plsc

NG = 256
NN = 64
NR = NN + 1
NF = 9
H = 768
L = 16
NC = 2
NS = 16
NW = NC * NS

GPW = NG // NW
SUB = 8
ROWS = SUB * NF
KPG = 9
GSTRIDE = 592
SMALL = 16
CB = H // L


def _body(idx9, ind_f, outd_f, atom_w, in_w, out_w, tok,
          out3, deg3,
          xidx, din, dout, bufA, bufB, obA, obB, tokv,
          semA, semB):
    wid = lax.axis_index("s") * NC + lax.axis_index("c")

    pltpu.sync_copy(idx9.at[pl.ds(wid * GPW * GSTRIDE, GPW * GSTRIDE)], xidx)
    pltpu.sync_copy(ind_f.at[pl.ds(wid * GPW * NN, GPW * NN)], din)
    pltpu.sync_copy(outd_f.at[pl.ds(wid * GPW * NN, GPW * NN)], dout)
    pltpu.sync_copy(tok, tokv)

    def gsize(k):
        return ROWS if k < 8 else SMALL

    def fire(gi, k, buf, sem):
        n = gsize(k)
        pltpu.async_copy(
            atom_w.at[xidx.at[pl.ds(gi * GSTRIDE + k * ROWS, n)]],
            buf.at[pl.ds(0, n)], sem)

    def wait_g(gi, k, buf, sem):
        n = gsize(k)
        pltpu.make_async_copy(
            atom_w.at[xidx.at[pl.ds(gi * GSTRIDE + k * ROWS, n)]],
            buf.at[pl.ds(0, n)], sem).wait()

    def compute(k, buf, ob):
        nrows = SUB if k < 8 else 1

        def jbody(j, _):
            c = j * L
            for n in range(nrows):
                if k == 0 and n == 0:
                    ob[0, pl.ds(c, L)] = tokv[0, pl.ds(c, L)]
                else:
                    base = n * NF
                    acc = buf[base, pl.ds(c, L)]
                    for f in range(1, NF):
                        acc = acc + buf[base + f, pl.ds(c, L)]
                    ob[n, pl.ds(c, L)] = acc
            return 0

        lax.fori_loop(0, CB, jbody, 0)

    def store(gi, k, ob):
        g = wid * GPW + gi
        if k < 8:
            pltpu.sync_copy(ob, out3.at[g, pl.ds(k * SUB, SUB), :])
        else:
            pltpu.sync_copy(ob.at[pl.ds(0, 1)], out3.at[g, pl.ds(NN, 1), :])

    fire(0, 0, bufA, semA)

    def pair_body(gp, _):
        gi0 = gp * 2
        for m in range(2 * KPG):
            gi = gi0 + m // KPG
            k = m % KPG
            buf, sem, ob = (bufA, semA, obA) if m % 2 == 0 else (bufB, semB, obB)
            nbuf, nsem = (bufB, semB) if m % 2 == 0 else (bufA, semA)
            wait_g(gi, k, buf, sem)
            if m == 2 * KPG - 1:
                @pl.when(gi0 + 2 < GPW)
                def _():
                    fire(gi0 + 2, 0, nbuf, nsem)
            elif k == KPG - 1:
                fire(gi + 1, 0, nbuf, nsem)
            else:
                fire(gi, k + 1, nbuf, nsem)
            compute(k, buf, ob)
            store(gi, k, ob)
        return 0

    lax.fori_loop(0, GPW // 2, pair_body, 0)

    def dbody(c, _):
        pltpu.async_copy(in_w.at[din.at[pl.ds(c * NN, NN)]],
                         bufA.at[pl.ds(0, NN)], semA)
        pltpu.async_copy(out_w.at[dout.at[pl.ds(c * NN, NN)]],
                         bufB.at[pl.ds(0, NN)], semB)
        pltpu.make_async_copy(in_w.at[din.at[pl.ds(c * NN, NN)]],
                              bufA.at[pl.ds(0, NN)], semA).wait()
        pltpu.make_async_copy(out_w.at[dout.at[pl.ds(c * NN, NN)]],
                              bufB.at[pl.ds(0, NN)], semB).wait()

        def rbody(r, _):
            for j in range(CB):
                col = j * L
                bufA[r, pl.ds(col, L)] = (bufA[r, pl.ds(col, L)]
                                          + bufB[r, pl.ds(col, L)])
            return 0

        lax.fori_loop(0, NN, rbody, 0)
        pltpu.sync_copy(bufA.at[pl.ds(0, NN)], deg3.at[wid * GPW + c])
        return 0

    lax.fori_loop(0, GPW, dbody, 0)


@jax.jit
def kernel(x, in_degree, out_degree, atom_w, in_w, out_w, graph_token):
    x3 = x.astype(jnp.int32).reshape(NG, NN, NF)
    per_g = jnp.concatenate(
        [jnp.zeros((NG, 1, NF), jnp.int32), x3], axis=1).reshape(NG, NR * NF)
    idx9 = jnp.pad(per_g, ((0, 0), (0, GSTRIDE - NR * NF))).reshape(-1)
    ind_f = in_degree.astype(jnp.int32).reshape(NG * NN)
    outd_f = out_degree.astype(jnp.int32).reshape(NG * NN)

    kfn = pl.kernel(
        _body,
        out_type=(
            jax.ShapeDtypeStruct((NG, NR, H), jnp.float32),
            jax.ShapeDtypeStruct((NG, NN, H), jnp.float32),
        ),
        mesh=plsc.VectorSubcoreMesh(core_axis_name="c", subcore_axis_name="s"),
        scratch_types=[
            pltpu.VMEM((GPW * GSTRIDE,), jnp.int32),
            pltpu.VMEM((GPW * NN,), jnp.int32),
            pltpu.VMEM((GPW * NN,), jnp.int32),
            pltpu.VMEM((ROWS, H), jnp.float32),
            pltpu.VMEM((ROWS, H), jnp.float32),
            pltpu.VMEM((SUB, H), jnp.float32),
            pltpu.VMEM((SUB, H), jnp.float32),
            pltpu.VMEM((1, H), jnp.float32),
            pltpu.SemaphoreType.DMA,
            pltpu.SemaphoreType.DMA,
        ],
    )
    return kfn(idx9, ind_f, outd_f, atom_w, in_w, out_w, graph_token)
